# baseline (speedup 1.0000x reference)
import functools

import jax
import jax.numpy as jnp
from jax import lax
from jax.experimental import pallas as pl
from jax.experimental.pallas import tpu as pltpu
from jax.experimental.pallas import tpu_sc as plsc


def _gather_sc(table2, idx_half):
    B = idx_half.shape[0]
    _, E2 = table2.shape
    info = plsc.get_sparse_core_info()
    nw = info.num_cores * info.num_subcores
    b_per_w = B // nw
    mesh = plsc.VectorSubcoreMesh(core_axis_name="c", subcore_axis_name="s")

    @functools.partial(
        pl.kernel,
        mesh=mesh,
        out_type=jax.ShapeDtypeStruct((B, E2), jnp.float32),
        scratch_types=[
            pltpu.VMEM((b_per_w,), jnp.int32),
            pltpu.VMEM((b_per_w, E2), jnp.float32),
            pltpu.SemaphoreType.DMA,
        ],
    )
    def gather_kernel(table_hbm, idx_hbm, out_hbm, idx_v, rows_v, sem):
        wid = lax.axis_index("s") * info.num_cores + lax.axis_index("c")
        base = wid * b_per_w
        pltpu.sync_copy(idx_hbm.at[pl.ds(base, b_per_w)], idx_v)
        pltpu.async_copy(table_hbm.at[idx_v], rows_v, sem).wait()
        pltpu.sync_copy(rows_v, out_hbm.at[pl.ds(base, b_per_w)])

    return gather_kernel(table2, idx_half)


def _matmul_body(h2_ref, par_ref, wt_hbm, o_ref, wbuf, sem_w):
    i = pl.program_id(0)

    @pl.when(i == 0)
    def _():
        pltpu.make_async_copy(wt_hbm, wbuf, sem_w).start()
        pltpu.make_async_copy(wt_hbm, wbuf, sem_w).wait()

    h2 = h2_ref[...]
    E = h2.shape[1] // 2
    hidden = jnp.where(par_ref[...] == 0, h2[:, :E], h2[:, E:])
    o_ref[...] = lax.dot_general(
        hidden,
        wbuf[...],
        (((1,), (0,)), ((), ())),
        preferred_element_type=jnp.float32,
    )


def _project(hidden2, parity, expand_w, mb=32):
    B = hidden2.shape[0]
    V, E = expand_w.shape
    w_t = expand_w.T
    return pl.pallas_call(
        _matmul_body,
        grid=(B // mb,),
        in_specs=[
            pl.BlockSpec((mb, 2 * E), lambda i: (i, 0)),
            pl.BlockSpec((mb, 1), lambda i: (i, 0)),
            pl.BlockSpec(memory_space=pltpu.HBM),
        ],
        out_specs=pl.BlockSpec((mb, V), lambda i: (i, 0)),
        out_shape=jax.ShapeDtypeStruct((B, V), jnp.float32),
        scratch_shapes=[
            pltpu.VMEM((E, V), jnp.float32),
            pltpu.SemaphoreType.DMA,
        ],
    )(hidden2, parity, w_t)


def kernel(input, embed_table, expand_w):
    V, E = embed_table.shape
    idx = input.astype(jnp.int32)
    table2 = embed_table.reshape(V // 2, 2 * E)
    hidden2 = _gather_sc(table2, idx // 2)
    parity = (idx & 1).reshape(-1, 1)
    return _project(hidden2, parity, expand_w)


# speedup vs baseline: 3.1968x; 3.1968x over previous
import functools

import jax
import jax.numpy as jnp
from jax import lax
from jax.experimental import pallas as pl
from jax.experimental.pallas import tpu as pltpu
from jax.experimental.pallas import tpu_sc as plsc


def _gather_sc(table2, idx_half):
    B = idx_half.shape[0]
    _, E2 = table2.shape
    info = plsc.get_sparse_core_info()
    nw = info.num_cores * info.num_subcores
    b_per_w = B // nw
    mesh = plsc.VectorSubcoreMesh(core_axis_name="c", subcore_axis_name="s")

    @functools.partial(
        pl.kernel,
        mesh=mesh,
        out_type=jax.ShapeDtypeStruct((B, E2), jnp.float32),
        scratch_types=[
            pltpu.VMEM((b_per_w,), jnp.int32),
            pltpu.VMEM((b_per_w, E2), jnp.float32),
            pltpu.SemaphoreType.DMA,
        ],
    )
    def gather_kernel(table_hbm, idx_hbm, out_hbm, idx_v, rows_v, sem):
        wid = lax.axis_index("s") * info.num_cores + lax.axis_index("c")
        base = wid * b_per_w
        pltpu.sync_copy(idx_hbm.at[pl.ds(base, b_per_w)], idx_v)
        pltpu.async_copy(table_hbm.at[idx_v], rows_v, sem).wait()
        pltpu.sync_copy(rows_v, out_hbm.at[pl.ds(base, b_per_w)])

    return gather_kernel(table2, idx_half)


def _matmul_body(h2_ref, par_ref, w_ref, o_ref, hbuf):
    j = pl.program_id(0)

    @pl.when(j == 0)
    def _():
        h2 = h2_ref[...]
        E = h2.shape[1] // 2
        hbuf[...] = jnp.where(par_ref[...] == 0, h2[:, :E], h2[:, E:])

    o_ref[...] = lax.dot_general(
        w_ref[...],
        hbuf[...],
        (((1,), (1,)), ((), ())),
        preferred_element_type=jnp.float32,
    )


def _project(hidden2, parity, expand_w, vs=800):
    B = hidden2.shape[0]
    V, E = expand_w.shape
    out_t = pl.pallas_call(
        _matmul_body,
        grid=(V // vs,),
        in_specs=[
            pl.BlockSpec((B, 2 * E), lambda j: (0, 0)),
            pl.BlockSpec((B, 1), lambda j: (0, 0)),
            pl.BlockSpec((vs, E), lambda j: (j, 0)),
        ],
        out_specs=pl.BlockSpec((vs, B), lambda j: (j, 0)),
        out_shape=jax.ShapeDtypeStruct((V, B), jnp.float32),
        scratch_shapes=[
            pltpu.VMEM((B, E), jnp.float32),
        ],
    )(hidden2, parity, expand_w)
    return out_t.T


def kernel(input, embed_table, expand_w):
    V, E = embed_table.shape
    idx = input.astype(jnp.int32)
    table2 = embed_table.reshape(V // 2, 2 * E)
    hidden2 = _gather_sc(table2, idx // 2)
    parity = (idx & 1).reshape(-1, 1)
    return _project(hidden2, parity, expand_w)
